# baseline (speedup 1.0000x reference)
import functools

import jax
import jax.numpy as jnp
from jax import lax
from jax.experimental import pallas as pl
from jax.experimental.pallas import tpu as pltpu
from jax.experimental.pallas import tpu_sc as plsc

VOCAB = 1000000
EMB = 64
VEC = 128
CLS = 100
B = 16384
L = 20
N = B * L

RB = 2048
QB = 249984
NRB = 123
VP4 = NRB * RB
_LASTW = 3 * QB + (NRB - 1) * RB
_TAIL = VOCAB - EMB

_NC = 2
_NS = 16
NW = _NC * _NS
PER_W = N // NW
STEP = 128
NSTEP = PER_W // STEP
TILE_ROWS = 512
K = TILE_ROWS // STEP
NOUT = PER_W // TILE_ROWS


def _pack_body(tt_ref, tail_ref, out_ref, buf_ref, sem_ref):
    i = pl.program_id(0)

    def start(step, slot):
        for q in range(4):
            if q < 3:
                src = pl.multiple_of(q * QB + step * RB, 128)
                pltpu.make_async_copy(
                    tt_ref.at[:, pl.ds(src, RB)], buf_ref.at[slot, q],
                    sem_ref.at[slot, q]).start()
            else:
                @pl.when(step < NRB - 1)
                def _():
                    src = pl.multiple_of(3 * QB + step * RB, 128)
                    pltpu.make_async_copy(
                        tt_ref.at[:, pl.ds(src, RB)], buf_ref.at[slot, 3],
                        sem_ref.at[slot, 3]).start()

                @pl.when(step == NRB - 1)
                def _():
                    pltpu.make_async_copy(
                        tt_ref.at[:, pl.ds(_LASTW, 128)],
                        buf_ref.at[slot, 3, :, pl.ds(0, 128)],
                        sem_ref.at[slot, 3]).start()

    def wait(step, slot):
        for q in range(3):
            pltpu.make_async_copy(
                tt_ref.at[:, pl.ds(0, RB)], buf_ref.at[slot, q],
                sem_ref.at[slot, q]).wait()

        @pl.when(step < NRB - 1)
        def _():
            pltpu.make_async_copy(
                tt_ref.at[:, pl.ds(0, RB)], buf_ref.at[slot, 3],
                sem_ref.at[slot, 3]).wait()

        @pl.when(step == NRB - 1)
        def _():
            pltpu.make_async_copy(
                tt_ref.at[:, pl.ds(0, 128)],
                buf_ref.at[slot, 3, :, pl.ds(0, 128)],
                sem_ref.at[slot, 3]).wait()

    @pl.when(i == 0)
    def _():
        start(0, 0)

    @pl.when(i + 1 < NRB)
    def _():
        start(i + 1, (i + 1) % 2)

    slot = i % 2
    wait(i, slot)
    q3 = buf_ref[slot, 3]
    q3_fix = jnp.concatenate(
        [q3[:, :128], tail_ref[...], q3[:, 192:]], axis=1)
    q3_use = jnp.where(i == NRB - 1, q3_fix, q3)
    x01 = jnp.concatenate(
        [jnp.transpose(buf_ref[slot, 0]), jnp.transpose(buf_ref[slot, 1])],
        axis=1)
    x23 = jnp.concatenate(
        [jnp.transpose(buf_ref[slot, 2]), jnp.transpose(q3_use)], axis=1)
    u = jax.lax.bitcast_convert_type(x01, jnp.int32)
    v = jax.lax.bitcast_convert_type(x23, jnp.int32)
    ur = u + (jnp.int32(0x7FFF) + ((u >> 16) & jnp.int32(1)))
    vr = v + (jnp.int32(0x7FFF) + ((v >> 16) & jnp.int32(1)))
    out_ref[...] = (vr & jnp.int32(-65536)) | ((ur >> 16) & jnp.int32(0xFFFF))


def _pack_table(table_t):
    tail = lax.slice(table_t, (0, _TAIL), (EMB, VOCAB))
    return pl.pallas_call(
        _pack_body,
        grid=(NRB,),
        in_specs=[
            pl.BlockSpec(memory_space=pl.ANY),
            pl.BlockSpec((EMB, EMB), lambda i: (0, 0)),
        ],
        out_specs=pl.BlockSpec((RB, 2 * EMB), lambda i: (i, 0)),
        out_shape=jax.ShapeDtypeStruct((VP4, 2 * EMB), jnp.int32),
        scratch_shapes=[
            pltpu.VMEM((2, 4, EMB, RB), jnp.float32),
            pltpu.SemaphoreType.DMA((2, 4)),
        ],
    )(table_t, tail)


def _sc_gather(idx, table4):
    mesh = plsc.VectorSubcoreMesh(core_axis_name="c", subcore_axis_name="s")

    @functools.partial(
        pl.kernel,
        out_type=jax.ShapeDtypeStruct((N, 2 * EMB), jnp.int32),
        mesh=mesh,
        scratch_types=[
            pltpu.VMEM((NSTEP, STEP), jnp.int32),
            pltpu.VMEM((TILE_ROWS, 2 * EMB), jnp.int32),
            pltpu.SemaphoreType.DMA,
        ],
    )
    def gather_kernel(idx_hbm, table_hbm, out_hbm, idx_v, rows_v, sem):
        wid = lax.axis_index("s") * _NC + lax.axis_index("c")
        pltpu.sync_copy(idx_hbm.at[wid], idx_v)
        base = wid * PER_W

        def drain_iter(g, carry):
            copies = [
                pltpu.async_copy(
                    table_hbm.at[idx_v.at[g * K + j]],
                    rows_v.at[pl.ds(j * STEP, STEP)],
                    sem,
                )
                for j in range(K)
            ]
            for c in copies:
                c.wait()
            pltpu.sync_copy(
                rows_v, out_hbm.at[pl.ds(base + g * TILE_ROWS, TILE_ROWS)]
            )
            return carry

        lax.fori_loop(0, NOUT, drain_iter, 0)

    return gather_kernel(idx, table4)


BLK = 2048
NBLK_B = B // BLK


def _mlp_body(emb_ref, par_ref, w1t_ref, b1_ref, w2t_ref, b2_ref,
              wct_ref, bc_ref, wt1t_ref, bt1_ref, wt2t_ref, bt2_ref,
              cls_ref, conf_ref, h_ref):
    xt32 = jnp.transpose(emb_ref[...])
    x01 = jax.lax.bitcast_convert_type(xt32 << 16, jnp.float32)
    x23 = jax.lax.bitcast_convert_type(xt32 & jnp.int32(-65536), jnp.float32)
    p = par_ref[0]
    sel01 = jnp.where(p == 0.0, x01[:EMB], x01[EMB:])
    sel23 = jnp.where(p == 2.0, x23[:EMB], x23[EMB:])
    x = jnp.where(p < 2.0, sel01, sel23)
    h1 = jnp.maximum(
        jnp.dot(w1t_ref[...], x, preferred_element_type=jnp.float32)
        + b1_ref[...], 0.0)
    h = jnp.maximum(
        jnp.dot(w2t_ref[...], h1, preferred_element_type=jnp.float32)
        + b2_ref[...], 0.0)
    h_ref[0] = h
    cls_ref[0] = (
        jnp.dot(wct_ref[...], h, preferred_element_type=jnp.float32)
        + bc_ref[...])
    t = jnp.maximum(
        jnp.dot(wt1t_ref[...], h, preferred_element_type=jnp.float32)
        + bt1_ref[...], 0.0)
    z = (jnp.dot(wt2t_ref[...], t, preferred_element_type=jnp.float32)
         + bt2_ref[...])
    ez = jnp.exp(-jnp.abs(z))
    conf_ref[0] = jnp.where(z >= 0, 1.0 / (1.0 + ez), ez / (1.0 + ez))


def _mlp(emb2, par, W1T, b1, W2T, b2, WcT, bc, Wt1T, bt1, Wt2T, bt2):
    full = lambda shape: pl.BlockSpec(shape, lambda l, j: (0, 0))
    return pl.pallas_call(
        _mlp_body,
        grid=(L, NBLK_B),
        in_specs=[
            pl.BlockSpec((BLK, 2 * EMB), lambda l, j: (l * NBLK_B + j, 0)),
            pl.BlockSpec((1, 1, BLK), lambda l, j: (l * NBLK_B + j, 0, 0)),
            full((VEC, EMB)), full((VEC, 1)),
            full((VEC // 2, VEC)), full((VEC // 2, 1)),
            full((CLS, VEC // 2)), full((CLS, 1)),
            full((VEC // 4, VEC // 2)), full((VEC // 4, 1)),
            full((1, VEC // 4)), full((1, 1)),
        ],
        out_specs=[
            pl.BlockSpec((1, CLS, BLK), lambda l, j: (l, 0, j)),
            pl.BlockSpec((1, 1, BLK), lambda l, j: (l, 0, j)),
            pl.BlockSpec((1, EMB, BLK), lambda l, j: (l, 0, j)),
        ],
        out_shape=[
            jax.ShapeDtypeStruct((L, CLS, B), jnp.float32),
            jax.ShapeDtypeStruct((L, 1, B), jnp.float32),
            jax.ShapeDtypeStruct((L, EMB, B), jnp.float32),
        ],
    )(emb2, par, W1T, b1.reshape(VEC, 1), W2T, b2.reshape(VEC // 2, 1),
      WcT, bc.reshape(CLS, 1), Wt1T, bt1.reshape(VEC // 4, 1),
      Wt2T, bt2.reshape(1, 1))


def kernel(input_label, table, W1, b1, W2, b2, Wc, bc, Wt1, bt1, Wt2, bt2):
    flat = jnp.transpose(input_label.astype(jnp.int32)).reshape(N)
    q = jnp.minimum(flat // QB, 3)
    idx = (flat - q * QB).reshape(NW, NSTEP, STEP)
    par = q.astype(jnp.float32).reshape(N // BLK, 1, BLK)
    table4 = _pack_table(jnp.transpose(table))
    emb2 = _sc_gather(idx, table4)
    clsT, confT, hT = _mlp(emb2, par, W1.T, b1, W2.T, b2, Wc.T, bc,
                           Wt1.T, bt1, Wt2.T, bt2)
    class_pred = jnp.transpose(clsT, (2, 0, 1))
    confidence = jnp.transpose(confT, (2, 0, 1))
    h = jnp.transpose(hT, (2, 0, 1))
    return (class_pred, confidence, h)


# speedup vs baseline: 1.1510x; 1.1510x over previous
import functools

import jax
import jax.numpy as jnp
from jax import lax
from jax.experimental import pallas as pl
from jax.experimental.pallas import tpu as pltpu
from jax.experimental.pallas import tpu_sc as plsc

VOCAB = 1000000
EMB = 64
VEC = 128
CLS = 100
B = 16384
L = 20
N = B * L

RB = 4096
QB = 249984
NRB = 62
VP4 = NRB * RB
_LASTW = 3 * QB + (NRB - 1) * RB
_TAIL = VOCAB - EMB

_NC = 2
_NS = 16
NW = _NC * _NS
PER_W = N // NW
STEP = 128
NSTEP = PER_W // STEP
TILE_ROWS = 512
K = TILE_ROWS // STEP
NOUT = PER_W // TILE_ROWS


def _pack_body(tt_ref, tail_ref, out_ref, buf_ref, sem_ref):
    i = pl.program_id(0)

    def start(step, slot):
        for q in range(4):
            if q < 3:
                src = pl.multiple_of(q * QB + step * RB, 128)
                pltpu.make_async_copy(
                    tt_ref.at[:, pl.ds(src, RB)], buf_ref.at[slot, q],
                    sem_ref.at[slot, q]).start()
            else:
                @pl.when(step < NRB - 1)
                def _():
                    src = pl.multiple_of(3 * QB + step * RB, 128)
                    pltpu.make_async_copy(
                        tt_ref.at[:, pl.ds(src, RB)], buf_ref.at[slot, 3],
                        sem_ref.at[slot, 3]).start()

                @pl.when(step == NRB - 1)
                def _():
                    pltpu.make_async_copy(
                        tt_ref.at[:, pl.ds(_LASTW, 128)],
                        buf_ref.at[slot, 3, :, pl.ds(0, 128)],
                        sem_ref.at[slot, 3]).start()

    def wait(step, slot):
        for q in range(3):
            pltpu.make_async_copy(
                tt_ref.at[:, pl.ds(0, RB)], buf_ref.at[slot, q],
                sem_ref.at[slot, q]).wait()

        @pl.when(step < NRB - 1)
        def _():
            pltpu.make_async_copy(
                tt_ref.at[:, pl.ds(0, RB)], buf_ref.at[slot, 3],
                sem_ref.at[slot, 3]).wait()

        @pl.when(step == NRB - 1)
        def _():
            pltpu.make_async_copy(
                tt_ref.at[:, pl.ds(0, 128)],
                buf_ref.at[slot, 3, :, pl.ds(0, 128)],
                sem_ref.at[slot, 3]).wait()

    @pl.when(i == 0)
    def _():
        start(0, 0)

    @pl.when(i + 1 < NRB)
    def _():
        start(i + 1, (i + 1) % 2)

    slot = i % 2
    wait(i, slot)
    q3 = buf_ref[slot, 3]
    q3_fix = jnp.concatenate(
        [q3[:, :128], tail_ref[...], q3[:, 192:]], axis=1)
    q3_use = jnp.where(i == NRB - 1, q3_fix, q3)
    x01 = jnp.concatenate(
        [jnp.transpose(buf_ref[slot, 0]), jnp.transpose(buf_ref[slot, 1])],
        axis=1)
    x23 = jnp.concatenate(
        [jnp.transpose(buf_ref[slot, 2]), jnp.transpose(q3_use)], axis=1)
    u = jax.lax.bitcast_convert_type(x01, jnp.int32)
    v = jax.lax.bitcast_convert_type(x23, jnp.int32)
    ur = u + (jnp.int32(0x7FFF) + ((u >> 16) & jnp.int32(1)))
    vr = v + (jnp.int32(0x7FFF) + ((v >> 16) & jnp.int32(1)))
    out_ref[...] = (vr & jnp.int32(-65536)) | ((ur >> 16) & jnp.int32(0xFFFF))


def _pack_table(table_t):
    tail = lax.slice(table_t, (0, _TAIL), (EMB, VOCAB))
    return pl.pallas_call(
        _pack_body,
        grid=(NRB,),
        in_specs=[
            pl.BlockSpec(memory_space=pl.ANY),
            pl.BlockSpec((EMB, EMB), lambda i: (0, 0)),
        ],
        out_specs=pl.BlockSpec((RB, 2 * EMB), lambda i: (i, 0)),
        out_shape=jax.ShapeDtypeStruct((VP4, 2 * EMB), jnp.int32),
        scratch_shapes=[
            pltpu.VMEM((2, 4, EMB, RB), jnp.float32),
            pltpu.SemaphoreType.DMA((2, 4)),
        ],
    )(table_t, tail)


def _sc_gather(idx, table4):
    mesh = plsc.VectorSubcoreMesh(core_axis_name="c", subcore_axis_name="s")

    @functools.partial(
        pl.kernel,
        out_type=jax.ShapeDtypeStruct((N, 2 * EMB), jnp.int32),
        mesh=mesh,
        scratch_types=[
            pltpu.VMEM((NSTEP, STEP), jnp.int32),
            pltpu.VMEM((TILE_ROWS, 2 * EMB), jnp.int32),
            pltpu.SemaphoreType.DMA,
        ],
    )
    def gather_kernel(idx_hbm, table_hbm, out_hbm, idx_v, rows_v, sem):
        wid = lax.axis_index("s") * _NC + lax.axis_index("c")
        pltpu.sync_copy(idx_hbm.at[wid], idx_v)
        base = wid * PER_W

        def drain_iter(g, carry):
            copies = [
                pltpu.async_copy(
                    table_hbm.at[idx_v.at[g * K + j]],
                    rows_v.at[pl.ds(j * STEP, STEP)],
                    sem,
                )
                for j in range(K)
            ]
            for c in copies:
                c.wait()
            pltpu.sync_copy(
                rows_v, out_hbm.at[pl.ds(base + g * TILE_ROWS, TILE_ROWS)]
            )
            return carry

        lax.fori_loop(0, NOUT, drain_iter, 0)

    return gather_kernel(idx, table4)


BLK = 4096
NBLK_B = B // BLK


def _mlp_body(emb_ref, par_ref, w1t_ref, b1_ref, w2t_ref, b2_ref,
              wct_ref, bc_ref, wt1t_ref, bt1_ref, wt2t_ref, bt2_ref,
              cls_ref, conf_ref, h_ref):
    xt32 = jnp.transpose(emb_ref[...])
    x01 = jax.lax.bitcast_convert_type(xt32 << 16, jnp.float32)
    x23 = jax.lax.bitcast_convert_type(xt32 & jnp.int32(-65536), jnp.float32)
    p = par_ref[0]
    sel01 = jnp.where(p == 0.0, x01[:EMB], x01[EMB:])
    sel23 = jnp.where(p == 2.0, x23[:EMB], x23[EMB:])
    x = jnp.where(p < 2.0, sel01, sel23)
    h1 = jnp.maximum(
        jnp.dot(w1t_ref[...], x, preferred_element_type=jnp.float32)
        + b1_ref[...], 0.0)
    h = jnp.maximum(
        jnp.dot(w2t_ref[...], h1, preferred_element_type=jnp.float32)
        + b2_ref[...], 0.0)
    h_ref[0] = h
    cls_ref[0] = (
        jnp.dot(wct_ref[...], h, preferred_element_type=jnp.float32)
        + bc_ref[...])
    t = jnp.maximum(
        jnp.dot(wt1t_ref[...], h, preferred_element_type=jnp.float32)
        + bt1_ref[...], 0.0)
    z = (jnp.dot(wt2t_ref[...], t, preferred_element_type=jnp.float32)
         + bt2_ref[...])
    ez = jnp.exp(-jnp.abs(z))
    conf_ref[0] = jnp.where(z >= 0, 1.0 / (1.0 + ez), ez / (1.0 + ez))


def _mlp(emb2, par, W1T, b1, W2T, b2, WcT, bc, Wt1T, bt1, Wt2T, bt2):
    full = lambda shape: pl.BlockSpec(shape, lambda l, j: (0, 0))
    return pl.pallas_call(
        _mlp_body,
        grid=(L, NBLK_B),
        in_specs=[
            pl.BlockSpec((BLK, 2 * EMB), lambda l, j: (l * NBLK_B + j, 0)),
            pl.BlockSpec((1, 1, BLK), lambda l, j: (l * NBLK_B + j, 0, 0)),
            full((VEC, EMB)), full((VEC, 1)),
            full((VEC // 2, VEC)), full((VEC // 2, 1)),
            full((CLS, VEC // 2)), full((CLS, 1)),
            full((VEC // 4, VEC // 2)), full((VEC // 4, 1)),
            full((1, VEC // 4)), full((1, 1)),
        ],
        out_specs=[
            pl.BlockSpec((1, CLS, BLK), lambda l, j: (l, 0, j)),
            pl.BlockSpec((1, 1, BLK), lambda l, j: (l, 0, j)),
            pl.BlockSpec((1, EMB, BLK), lambda l, j: (l, 0, j)),
        ],
        out_shape=[
            jax.ShapeDtypeStruct((L, CLS, B), jnp.float32),
            jax.ShapeDtypeStruct((L, 1, B), jnp.float32),
            jax.ShapeDtypeStruct((L, EMB, B), jnp.float32),
        ],
    )(emb2, par, W1T, b1.reshape(VEC, 1), W2T, b2.reshape(VEC // 2, 1),
      WcT, bc.reshape(CLS, 1), Wt1T, bt1.reshape(VEC // 4, 1),
      Wt2T, bt2.reshape(1, 1))


def kernel(input_label, table, W1, b1, W2, b2, Wc, bc, Wt1, bt1, Wt2, bt2):
    flat = jnp.transpose(input_label.astype(jnp.int32)).reshape(N)
    q = jnp.minimum(flat // QB, 3)
    idx = (flat - q * QB).reshape(NW, NSTEP, STEP)
    par = q.astype(jnp.float32).reshape(N // BLK, 1, BLK)
    table4 = _pack_table(jnp.transpose(table))
    emb2 = _sc_gather(idx, table4)
    clsT, confT, hT = _mlp(emb2, par, W1.T, b1, W2.T, b2, Wc.T, bc,
                           Wt1.T, bt1, Wt2.T, bt2)
    class_pred = jnp.transpose(clsT, (2, 0, 1))
    confidence = jnp.transpose(confT, (2, 0, 1))
    h = jnp.transpose(hT, (2, 0, 1))
    return (class_pred, confidence, h)


# speedup vs baseline: 1.2278x; 1.0667x over previous
import functools

import jax
import jax.numpy as jnp
from jax import lax
from jax.experimental import pallas as pl
from jax.experimental.pallas import tpu as pltpu
from jax.experimental.pallas import tpu_sc as plsc

VOCAB = 1000000
EMB = 64
VEC = 128
CLS = 100
B = 16384
L = 20
N = B * L

RB = 8192
QB = 249984
NRB = 31
VP4 = NRB * RB
_LASTW = 3 * QB + (NRB - 1) * RB
_TAIL = VOCAB - EMB
_W3 = _TAIL - _LASTW

_NC = 2
_NS = 16
NW = _NC * _NS
PER_W = N // NW
STEP = 128
NSTEP = PER_W // STEP
TILE_ROWS = 512
K = TILE_ROWS // STEP
NOUT = PER_W // TILE_ROWS


def _pack_body(tt_ref, tail_ref, out_ref, buf_ref, sem_ref):
    i = pl.program_id(0)

    def start(step, slot):
        for q in range(4):
            if q < 3:
                src = pl.multiple_of(q * QB + step * RB, 128)
                pltpu.make_async_copy(
                    tt_ref.at[:, pl.ds(src, RB)], buf_ref.at[slot, q],
                    sem_ref.at[slot, q]).start()
            else:
                @pl.when(step < NRB - 1)
                def _():
                    src = pl.multiple_of(3 * QB + step * RB, 128)
                    pltpu.make_async_copy(
                        tt_ref.at[:, pl.ds(src, RB)], buf_ref.at[slot, 3],
                        sem_ref.at[slot, 3]).start()

                @pl.when(step == NRB - 1)
                def _():
                    pltpu.make_async_copy(
                        tt_ref.at[:, pl.ds(_LASTW, _W3)],
                        buf_ref.at[slot, 3, :, pl.ds(0, _W3)],
                        sem_ref.at[slot, 3]).start()

    def wait(step, slot):
        for q in range(3):
            pltpu.make_async_copy(
                tt_ref.at[:, pl.ds(0, RB)], buf_ref.at[slot, q],
                sem_ref.at[slot, q]).wait()

        @pl.when(step < NRB - 1)
        def _():
            pltpu.make_async_copy(
                tt_ref.at[:, pl.ds(0, RB)], buf_ref.at[slot, 3],
                sem_ref.at[slot, 3]).wait()

        @pl.when(step == NRB - 1)
        def _():
            pltpu.make_async_copy(
                tt_ref.at[:, pl.ds(0, _W3)],
                buf_ref.at[slot, 3, :, pl.ds(0, _W3)],
                sem_ref.at[slot, 3]).wait()

    @pl.when(i == 0)
    def _():
        start(0, 0)

    @pl.when(i + 1 < NRB)
    def _():
        start(i + 1, (i + 1) % 2)

    slot = i % 2
    wait(i, slot)
    q3 = buf_ref[slot, 3]
    q3_fix = jnp.concatenate(
        [q3[:, :_W3], tail_ref[...], q3[:, _W3 + EMB:]], axis=1)
    q3_use = jnp.where(i == NRB - 1, q3_fix, q3)
    x01 = jnp.concatenate(
        [jnp.transpose(buf_ref[slot, 0]), jnp.transpose(buf_ref[slot, 1])],
        axis=1)
    x23 = jnp.concatenate(
        [jnp.transpose(buf_ref[slot, 2]), jnp.transpose(q3_use)], axis=1)
    u = jax.lax.bitcast_convert_type(x01, jnp.int32)
    v = jax.lax.bitcast_convert_type(x23, jnp.int32)
    ur = u + (jnp.int32(0x7FFF) + ((u >> 16) & jnp.int32(1)))
    vr = v + (jnp.int32(0x7FFF) + ((v >> 16) & jnp.int32(1)))
    out_ref[...] = (vr & jnp.int32(-65536)) | ((ur >> 16) & jnp.int32(0xFFFF))


def _pack_table(table_t):
    tail = lax.slice(table_t, (0, _TAIL), (EMB, VOCAB))
    return pl.pallas_call(
        _pack_body,
        grid=(NRB,),
        in_specs=[
            pl.BlockSpec(memory_space=pl.ANY),
            pl.BlockSpec((EMB, EMB), lambda i: (0, 0)),
        ],
        out_specs=pl.BlockSpec((RB, 2 * EMB), lambda i: (i, 0)),
        out_shape=jax.ShapeDtypeStruct((VP4, 2 * EMB), jnp.int32),
        scratch_shapes=[
            pltpu.VMEM((2, 4, EMB, RB), jnp.float32),
            pltpu.SemaphoreType.DMA((2, 4)),
        ],
    )(table_t, tail)


def _sc_gather(idx, table4):
    mesh = plsc.VectorSubcoreMesh(core_axis_name="c", subcore_axis_name="s")

    @functools.partial(
        pl.kernel,
        out_type=jax.ShapeDtypeStruct((N, 2 * EMB), jnp.int32),
        mesh=mesh,
        scratch_types=[
            pltpu.VMEM((NSTEP, STEP), jnp.int32),
            pltpu.VMEM((TILE_ROWS, 2 * EMB), jnp.int32),
            pltpu.SemaphoreType.DMA,
        ],
    )
    def gather_kernel(idx_hbm, table_hbm, out_hbm, idx_v, rows_v, sem):
        wid = lax.axis_index("s") * _NC + lax.axis_index("c")
        pltpu.sync_copy(idx_hbm.at[wid], idx_v)
        base = wid * PER_W

        def drain_iter(g, carry):
            copies = [
                pltpu.async_copy(
                    table_hbm.at[idx_v.at[g * K + j]],
                    rows_v.at[pl.ds(j * STEP, STEP)],
                    sem,
                )
                for j in range(K)
            ]
            for c in copies:
                c.wait()
            pltpu.sync_copy(
                rows_v, out_hbm.at[pl.ds(base + g * TILE_ROWS, TILE_ROWS)]
            )
            return carry

        lax.fori_loop(0, NOUT, drain_iter, 0)

    return gather_kernel(idx, table4)


BLK = 8192
NBLK_B = B // BLK


def _mlp_body(emb_ref, par_ref, w1t_ref, b1_ref, w2t_ref, b2_ref,
              wct_ref, bc_ref, wt1t_ref, bt1_ref, wt2t_ref, bt2_ref,
              cls_ref, conf_ref, h_ref):
    xt32 = jnp.transpose(emb_ref[...])
    x01 = jax.lax.bitcast_convert_type(xt32 << 16, jnp.float32)
    x23 = jax.lax.bitcast_convert_type(xt32 & jnp.int32(-65536), jnp.float32)
    p = par_ref[0]
    sel01 = jnp.where(p == 0.0, x01[:EMB], x01[EMB:])
    sel23 = jnp.where(p == 2.0, x23[:EMB], x23[EMB:])
    x = jnp.where(p < 2.0, sel01, sel23)
    h1 = jnp.maximum(
        jnp.dot(w1t_ref[...], x, preferred_element_type=jnp.float32)
        + b1_ref[...], 0.0)
    h = jnp.maximum(
        jnp.dot(w2t_ref[...], h1, preferred_element_type=jnp.float32)
        + b2_ref[...], 0.0)
    h_ref[0] = h
    cls_ref[0] = (
        jnp.dot(wct_ref[...], h, preferred_element_type=jnp.float32)
        + bc_ref[...])
    t = jnp.maximum(
        jnp.dot(wt1t_ref[...], h, preferred_element_type=jnp.float32)
        + bt1_ref[...], 0.0)
    z = (jnp.dot(wt2t_ref[...], t, preferred_element_type=jnp.float32)
         + bt2_ref[...])
    ez = jnp.exp(-jnp.abs(z))
    conf_ref[0] = jnp.where(z >= 0, 1.0 / (1.0 + ez), ez / (1.0 + ez))


def _mlp(emb2, par, W1T, b1, W2T, b2, WcT, bc, Wt1T, bt1, Wt2T, bt2):
    full = lambda shape: pl.BlockSpec(shape, lambda l, j: (0, 0))
    return pl.pallas_call(
        _mlp_body,
        grid=(L, NBLK_B),
        in_specs=[
            pl.BlockSpec((BLK, 2 * EMB), lambda l, j: (l * NBLK_B + j, 0)),
            pl.BlockSpec((1, 1, BLK), lambda l, j: (l * NBLK_B + j, 0, 0)),
            full((VEC, EMB)), full((VEC, 1)),
            full((VEC // 2, VEC)), full((VEC // 2, 1)),
            full((CLS, VEC // 2)), full((CLS, 1)),
            full((VEC // 4, VEC // 2)), full((VEC // 4, 1)),
            full((1, VEC // 4)), full((1, 1)),
        ],
        out_specs=[
            pl.BlockSpec((1, CLS, BLK), lambda l, j: (l, 0, j)),
            pl.BlockSpec((1, 1, BLK), lambda l, j: (l, 0, j)),
            pl.BlockSpec((1, EMB, BLK), lambda l, j: (l, 0, j)),
        ],
        out_shape=[
            jax.ShapeDtypeStruct((L, CLS, B), jnp.float32),
            jax.ShapeDtypeStruct((L, 1, B), jnp.float32),
            jax.ShapeDtypeStruct((L, EMB, B), jnp.float32),
        ],
    )(emb2, par, W1T, b1.reshape(VEC, 1), W2T, b2.reshape(VEC // 2, 1),
      WcT, bc.reshape(CLS, 1), Wt1T, bt1.reshape(VEC // 4, 1),
      Wt2T, bt2.reshape(1, 1))


def kernel(input_label, table, W1, b1, W2, b2, Wc, bc, Wt1, bt1, Wt2, bt2):
    flat = jnp.transpose(input_label.astype(jnp.int32)).reshape(N)
    q = jnp.minimum(flat // QB, 3)
    idx = (flat - q * QB).reshape(NW, NSTEP, STEP)
    par = q.astype(jnp.float32).reshape(N // BLK, 1, BLK)
    table4 = _pack_table(jnp.transpose(table))
    emb2 = _sc_gather(idx, table4)
    clsT, confT, hT = _mlp(emb2, par, W1.T, b1, W2.T, b2, Wc.T, bc,
                           Wt1.T, bt1, Wt2.T, bt2)
    class_pred = jnp.transpose(clsT, (2, 0, 1))
    confidence = jnp.transpose(confT, (2, 0, 1))
    h = jnp.transpose(hT, (2, 0, 1))
    return (class_pred, confidence, h)
